# baseline (speedup 1.0000x reference)
import functools

import jax
import jax.numpy as jnp
from jax import lax
from jax.experimental import pallas as pl
from jax.experimental.pallas import tpu as pltpu
from jax.experimental.pallas import tpu_sc as plsc

VOCAB = 1000000
D = 64
SCALE = 8.0

NC = 2
NS = 16
NW = NC * NS

S = 200
BATCH = 4096
B2 = S * BATCH


VW = 256
NFULL = VOCAB // VW
TAIL = VOCAB - NFULL * VW
PA_BASE = NFULL // NW
PA_EXTRA = NFULL - PA_BASE * NW


def _prep_body(tt_hbm, tail_hbm, t2_hbm, in0, in1, st0, st1, isem0, isem1,
               osem0, osem1):
    wid = lax.axis_index("s") * NC + lax.axis_index("c")
    ntask = PA_BASE + jnp.where(wid < PA_EXTRA, 1, 0)
    g0 = wid * PA_BASE + jnp.minimum(wid, PA_EXTRA)
    iota = lax.iota(jnp.int32, 16)

    ins = (in0, in1)
    sts = (st0, st1)
    isems = (isem0, isem1)
    osems = (osem0, osem1)

    def in_start(i, inb, isem):
        v0 = (g0 + i) * VW
        for k in range(VW // 128):
            pltpu.make_async_copy(
                tt_hbm.at[:, pl.ds(v0 + k * 128, 128)], inb.at[k], isem
            ).start()

    def in_wait(i, inb, isem):
        v0 = (g0 + i) * VW
        for k in range(VW // 128):
            pltpu.make_async_copy(
                tt_hbm.at[:, pl.ds(v0 + k * 128, 128)], inb.at[k], isem
            ).wait()

    def out_start(i, st, osem):
        v0 = (g0 + i) * VW
        pltpu.make_async_copy(st, t2_hbm.at[pl.ds(v0, VW)], osem).start()

    def transpose(inb, st):
        @plsc.parallel_loop(0, (VW // 128) * D * 8, unroll=8)
        def _(q):
            k = q >> 9
            d = (q >> 3) & (D - 1)
            m = q & 7
            v = inb[k, d, pl.ds(m * 16, 16)] * SCALE
            row = iota + (k * 128 + m * 16)
            col = jnp.full((16,), 1, jnp.int32) * d
            plsc.store_scatter(st, [row, col], v)

    in_start(0, in0, isem0)
    in_start(1, in1, isem1)

    def outer(o, carry):
        for b in (0, 1):
            i = o * 2 + b

            @pl.when(i < ntask)
            def _():
                in_wait(i, ins[b], isems[b])

                @pl.when(i >= 2)
                def _():
                    pltpu.make_async_copy(
                        sts[b], t2_hbm.at[pl.ds(0, VW)], osems[b]).wait()

                transpose(ins[b], sts[b])
                out_start(i, sts[b], osems[b])

                @pl.when(i + 2 < ntask)
                def _():
                    in_start(i + 2, ins[b], isems[b])

        return carry

    lax.fori_loop(0, (PA_BASE + 2) // 2, outer, 0)
    pltpu.make_async_copy(st0, t2_hbm.at[pl.ds(0, VW)], osem0).wait()
    pltpu.make_async_copy(st1, t2_hbm.at[pl.ds(0, VW)], osem1).wait()

    @pl.when(wid == NW - 1)
    def _():
        pltpu.sync_copy(tail_hbm, st0.at[pl.ds(0, TAIL)])
        pltpu.sync_copy(st0.at[pl.ds(0, TAIL)],
                        t2_hbm.at[pl.ds(NFULL * VW, TAIL)])


@jax.jit
def _prep(table_t, tail):
    mesh = plsc.VectorSubcoreMesh(core_axis_name="c", subcore_axis_name="s")
    k = functools.partial(
        pl.kernel,
        out_type=jax.ShapeDtypeStruct((VOCAB, 128), jnp.float32),
        mesh=mesh,
        scratch_types=[
            pltpu.VMEM((VW // 128, D, 128), jnp.float32),
            pltpu.VMEM((VW // 128, D, 128), jnp.float32),
            pltpu.VMEM((VW, 128), jnp.float32),
            pltpu.VMEM((VW, 128), jnp.float32),
            pltpu.SemaphoreType.DMA,
            pltpu.SemaphoreType.DMA,
            pltpu.SemaphoreType.DMA,
            pltpu.SemaphoreType.DMA,
        ],
        compiler_params=pltpu.CompilerParams(needs_layout_passes=False),
    )(_prep_body)
    return k(table_t, tail)


CB = 512
NTASK = B2 // CB
TPW = NTASK // NW
RPW = TPW * CB


def _emb_body(xt_hbm, table_hbm, o_hbm, idxall, rows0, rows1,
              gsem0, gsem1, osem0, osem1):
    wid = lax.axis_index("s") * NC + lax.axis_index("c")
    t0 = wid * TPW

    pltpu.sync_copy(xt_hbm.at[pl.ds(t0 * CB, RPW)], idxall)

    @plsc.parallel_loop(0, RPW // 16, unroll=8)
    def _(r):
        sl = pl.ds(r * 16, 16)
        idxall[sl] = idxall[sl] * 2

    def gather(i, rows_v, gsem):
        return pltpu.make_async_copy(
            table_hbm.at[idxall.at[pl.ds(i * CB, CB)]], rows_v, gsem)

    def out_copy(i, rows_v, osem):
        return pltpu.make_async_copy(
            rows_v, o_hbm.at[pl.ds((t0 + i) * CB, CB)], osem)

    gather(0, rows0, gsem0).start()
    gather(1, rows1, gsem1).start()

    rows = (rows0, rows1)
    gsems = (gsem0, gsem1)
    osems = (osem0, osem1)

    def outer(o, carry):
        for b in (0, 1):
            i = o * 2 + b
            gather(i, rows[b], gsems[b]).wait()
            out_copy(i, rows[b], osems[b]).start()

            @pl.when(i + 2 < TPW)
            def _():
                out_copy(i, rows[b], osems[b]).wait()
                gather(i + 2, rows[b], gsems[b]).start()

        return carry

    lax.fori_loop(0, TPW // 2, outer, 0)
    out_copy(TPW - 2, rows0, osem0).wait()
    out_copy(TPW - 1, rows1, osem1).wait()


@jax.jit
def _embedding(xt_flat, table2):
    mesh = plsc.VectorSubcoreMesh(core_axis_name="c", subcore_axis_name="s")
    k = functools.partial(
        pl.kernel,
        out_type=jax.ShapeDtypeStruct((B2, D), jnp.float32),
        mesh=mesh,
        scratch_types=[
            pltpu.VMEM((RPW,), jnp.int32),
            pltpu.VMEM((CB, D), jnp.float32),
            pltpu.VMEM((CB, D), jnp.float32),
            pltpu.SemaphoreType.DMA,
            pltpu.SemaphoreType.DMA,
            pltpu.SemaphoreType.DMA,
            pltpu.SemaphoreType.DMA,
        ],
        compiler_params=pltpu.CompilerParams(
            use_tc_tiling_on_sc=False, needs_layout_passes=False
        ),
    )(_emb_body)
    return k(xt_flat, table2)


def kernel(x, table):
    tail = jnp.pad(table[NFULL * VW:] * SCALE, ((0, 0), (0, 128 - D)))
    t2 = _prep(jnp.transpose(table), tail)
    t2v = t2.reshape(2 * VOCAB, D)
    xt = jnp.transpose(x).reshape(-1)
    o = _embedding(xt, t2v)
    o = o.reshape(S, BATCH, D)
    return jnp.transpose(o, (1, 0, 2))


# speedup vs baseline: 1.5006x; 1.5006x over previous
import functools

import jax
import jax.numpy as jnp
from jax import lax
from jax.experimental import pallas as pl
from jax.experimental.pallas import tpu as pltpu
from jax.experimental.pallas import tpu_sc as plsc

VOCAB = 1000000
D = 64
SCALE = 8.0

NC = 2
NS = 16
NW = NC * NS

S = 200
BATCH = 4096
B2 = S * BATCH

CB = 256
TW = CB + 1
CHUNKS_PER_S = BATCH // CB
NTASK = S * CHUNKS_PER_S
TPW = NTASK // NW
RPW = TPW * CB


def _emb_body(xt_hbm, table_hbm, o2_hbm, idxall, rows0, rows1, tr0, tr1,
              gsem0, gsem1, osem0, osem1):
    wid = lax.axis_index("s") * NC + lax.axis_index("c")
    t0 = wid * TPW
    iota = lax.iota(jnp.int32, 16)

    pltpu.sync_copy(xt_hbm.at[pl.ds(t0 * CB, RPW)], idxall)

    @plsc.parallel_loop(0, RPW // 16, unroll=8)
    def _(r):
        sl = pl.ds(r * 16, 16)
        idxall[sl] = idxall[sl] * 2

    def gather(i, rows_v, gsem):
        return pltpu.make_async_copy(
            table_hbm.at[idxall.at[pl.ds(i * CB, CB)]], rows_v, gsem)

    def out_copy(i, tr_v, osem):
        tt = t0 + i
        s_id = tt // CHUNKS_PER_S
        c_id = tt % CHUNKS_PER_S
        return pltpu.make_async_copy(
            tr_v.at[:, pl.ds(0, CB)],
            o2_hbm.at[s_id, :, pl.ds(c_id * CB, CB)], osem)

    def transpose(rows_v, tr_v):
        @plsc.parallel_loop(0, CB * (D // 16), unroll=8)
        def _(q):
            r = q >> 2
            u = q & (D // 16 - 1)
            v = rows_v[r, pl.ds(u * 16, 16)]
            row = iota + u * 16
            col = jnp.full((16,), 1, jnp.int32) * r
            plsc.store_scatter(tr_v, [row, col], v)

    gather(0, rows0, gsem0).start()
    gather(1, rows1, gsem1).start()

    rows = (rows0, rows1)
    trs = (tr0, tr1)
    gsems = (gsem0, gsem1)
    osems = (osem0, osem1)

    def outer(o, carry):
        for b in (0, 1):
            i = o * 2 + b
            gather(i, rows[b], gsems[b]).wait()

            @pl.when(i >= 2)
            def _():
                out_copy(i - 2, trs[b], osems[b]).wait()

            transpose(rows[b], trs[b])
            out_copy(i, trs[b], osems[b]).start()

            @pl.when(i + 2 < TPW)
            def _():
                gather(i + 2, rows[b], gsems[b]).start()

        return carry

    lax.fori_loop(0, TPW // 2, outer, 0)
    out_copy(TPW - 2, tr0, osem0).wait()
    out_copy(TPW - 1, tr1, osem1).wait()


@jax.jit
def _embedding(xt_flat, table2):
    mesh = plsc.VectorSubcoreMesh(core_axis_name="c", subcore_axis_name="s")
    k = functools.partial(
        pl.kernel,
        out_type=jax.ShapeDtypeStruct((S, D, BATCH), jnp.float32),
        mesh=mesh,
        scratch_types=[
            pltpu.VMEM((RPW,), jnp.int32),
            pltpu.VMEM((CB, D), jnp.float32),
            pltpu.VMEM((CB, D), jnp.float32),
            pltpu.VMEM((D, TW), jnp.float32),
            pltpu.VMEM((D, TW), jnp.float32),
            pltpu.SemaphoreType.DMA,
            pltpu.SemaphoreType.DMA,
            pltpu.SemaphoreType.DMA,
            pltpu.SemaphoreType.DMA,
        ],
        compiler_params=pltpu.CompilerParams(
            use_tc_tiling_on_sc=False, needs_layout_passes=False
        ),
    )(_emb_body)
    return k(xt_flat, table2)


def kernel(x, table):
    xt = jnp.transpose(x).reshape(-1)
    t2 = jnp.pad(table * SCALE, ((0, 0), (0, D))).reshape(2 * VOCAB, D)
    o2 = _embedding(xt, t2)
    return jnp.transpose(o2, (2, 0, 1))


# speedup vs baseline: 1.8362x; 1.2236x over previous
import functools

import jax
import jax.numpy as jnp
from jax import lax
from jax.experimental import pallas as pl
from jax.experimental.pallas import tpu as pltpu
from jax.experimental.pallas import tpu_sc as plsc

VOCAB = 1000000
D = 64
SCALE = 8.0

NC = 2
NS = 16
NW = NC * NS

S = 200
BATCH = 4096
B2 = S * BATCH

CB = 256
TW = CB + 1
CHUNKS_PER_S = BATCH // CB
NTASK = S * CHUNKS_PER_S
TPW = NTASK // NW
RPW = TPW * CB


def _emb_body(xt_hbm, table_hbm, o2_hbm, idxall, rows0, rows1, tr0, tr1,
              gsem0, gsem1, osem0, osem1):
    wid = lax.axis_index("s") * NC + lax.axis_index("c")
    t0 = wid * TPW
    iota = lax.iota(jnp.int32, 16)

    pltpu.sync_copy(xt_hbm.at[pl.ds(t0 * CB, RPW)], idxall)

    @plsc.parallel_loop(0, RPW // 16, unroll=8)
    def _(r):
        sl = pl.ds(r * 16, 16)
        idxall[sl] = idxall[sl] * 2

    def gather(i, rows_v, gsem):
        return pltpu.make_async_copy(
            table_hbm.at[idxall.at[pl.ds(i * CB, CB)]], rows_v, gsem)

    def out_copy(i, tr_v, osem):
        tt = t0 + i
        s_id = tt // CHUNKS_PER_S
        c_id = tt % CHUNKS_PER_S
        return pltpu.make_async_copy(
            tr_v.at[:, pl.ds(0, CB)],
            o2_hbm.at[s_id, :, pl.ds(c_id * CB, CB)], osem)

    def transpose(rows_v, tr_v):
        @plsc.parallel_loop(0, CB * (D // 16), unroll=8)
        def _(q):
            r = q >> 2
            u = q & (D // 16 - 1)
            v = rows_v[r, pl.ds(u * 16, 16)] * SCALE
            row = iota + u * 16
            col = jnp.full((16,), 1, jnp.int32) * r
            plsc.store_scatter(tr_v, [row, col], v)

    gather(0, rows0, gsem0).start()
    gather(1, rows1, gsem1).start()

    rows = (rows0, rows1)
    trs = (tr0, tr1)
    gsems = (gsem0, gsem1)
    osems = (osem0, osem1)

    def outer(o, carry):
        for b in (0, 1):
            i = o * 2 + b
            gather(i, rows[b], gsems[b]).wait()

            @pl.when(i >= 2)
            def _():
                out_copy(i - 2, trs[b], osems[b]).wait()

            transpose(rows[b], trs[b])
            out_copy(i, trs[b], osems[b]).start()

            @pl.when(i + 2 < TPW)
            def _():
                gather(i + 2, rows[b], gsems[b]).start()

        return carry

    lax.fori_loop(0, TPW // 2, outer, 0)
    out_copy(TPW - 2, tr0, osem0).wait()
    out_copy(TPW - 1, tr1, osem1).wait()


@jax.jit
def _embedding(xt_flat, table2):
    mesh = plsc.VectorSubcoreMesh(core_axis_name="c", subcore_axis_name="s")
    k = functools.partial(
        pl.kernel,
        out_type=jax.ShapeDtypeStruct((S, D, BATCH), jnp.float32),
        mesh=mesh,
        scratch_types=[
            pltpu.VMEM((RPW,), jnp.int32),
            pltpu.VMEM((CB, D), jnp.float32),
            pltpu.VMEM((CB, D), jnp.float32),
            pltpu.VMEM((D, TW), jnp.float32),
            pltpu.VMEM((D, TW), jnp.float32),
            pltpu.SemaphoreType.DMA,
            pltpu.SemaphoreType.DMA,
            pltpu.SemaphoreType.DMA,
            pltpu.SemaphoreType.DMA,
        ],
        compiler_params=pltpu.CompilerParams(
            use_tc_tiling_on_sc=False, needs_layout_passes=False
        ),
    )(_emb_body)
    return k(xt_flat, table2)


def kernel(x, table):
    xt = jnp.transpose(x).reshape(-1)
    t2 = jnp.pad(table, ((0, 0), (0, D))).reshape(2 * VOCAB, D)
    o2 = _embedding(xt, t2)
    return jnp.transpose(o2, (2, 0, 1))
